# baseline (speedup 1.0000x reference)
import jax
import jax.numpy as jnp
from jax.experimental import pallas as pl
from jax.experimental.pallas import tpu as pltpu

_TOP_K = 8
_RC = 128


def _topk_softmax_chunk_t(st, iota, ef):
    vals = []
    idxs = []
    for k in range(_TOP_K):
        m = jnp.max(st, axis=0, keepdims=True)
        eq = st == m
        hit = jnp.where(eq, iota, ef)
        idx = jnp.min(hit, axis=0, keepdims=True)
        vals.append(m)
        idxs.append(idx)
        if k + 1 < _TOP_K:
            st = jnp.where(eq, -jnp.inf, st)
    v = jnp.concatenate(vals, axis=0)
    ix = jnp.concatenate(idxs, axis=0)
    ex = jnp.exp(v - v[0:1, :])
    p = ex / jnp.sum(ex, axis=0, keepdims=True)
    return p.T, ix.T.astype(jnp.int32)


def _router_body(x_ref, w_ref, probs_ref, idx_ref):
    bt = x_ref.shape[1]
    e = w_ref.shape[0]
    w = w_ref[...]
    iota = jax.lax.broadcasted_iota(jnp.int32, (e, _RC), 0).astype(jnp.float32)
    ef = float(e)
    n = bt // _RC

    def dot_t(c):
        lo = c * _RC
        return jax.lax.dot_general(
            w, x_ref[0, lo:lo + _RC, :], (((1,), (1,)), ((), ())),
            preferred_element_type=jnp.float32)

    st_next = dot_t(0)
    for c in range(n):
        st = st_next
        if c + 1 < n:
            st_next = dot_t(c + 1)
        p, ix = _topk_softmax_chunk_t(st, iota, ef)
        probs_ref[0, c * _RC:(c + 1) * _RC, :] = p
        idx_ref[0, c * _RC:(c + 1) * _RC, :] = ix


def kernel(x, W_gate):
    b, s, d = x.shape
    e = W_gate.shape[0]
    t = b * s
    bt = min(512, s)
    grid = (t // bt,)
    spb = s // bt
    probs, idx = pl.pallas_call(
        _router_body,
        grid=grid,
        in_specs=[
            pl.BlockSpec((1, bt, d), lambda i: (i // spb, i % spb, 0)),
            pl.BlockSpec((e, d), lambda i: (0, 0)),
        ],
        out_specs=[
            pl.BlockSpec((1, bt, _TOP_K), lambda i: (i // spb, i % spb, 0)),
            pl.BlockSpec((1, bt, _TOP_K), lambda i: (i // spb, i % spb, 0)),
        ],
        out_shape=[
            jax.ShapeDtypeStruct((b, s, _TOP_K), jnp.float32),
            jax.ShapeDtypeStruct((b, s, _TOP_K), jnp.int32),
        ],
        compiler_params=pltpu.CompilerParams(
            dimension_semantics=("parallel",)),
    )(x, W_gate)
    return probs, idx


# speedup vs baseline: 1.0708x; 1.0708x over previous
import jax
import jax.numpy as jnp
from jax.experimental import pallas as pl
from jax.experimental.pallas import tpu as pltpu

_TOP_K = 8
_RC = 128


def _topk_softmax_chunk_t(st, iota, ef):
    vals = []
    idxs = []
    for k in range(_TOP_K):
        m = jnp.max(st, axis=0, keepdims=True)
        eq = st == m
        hit = jnp.where(eq, iota, ef)
        idx = jnp.min(hit, axis=0, keepdims=True)
        vals.append(m)
        idxs.append(idx)
        if k + 1 < _TOP_K:
            st = jnp.where(eq, -jnp.inf, st)
    v = jnp.concatenate(vals, axis=0)
    ix = jnp.concatenate(idxs, axis=0)
    ex = jnp.exp(v - v[0:1, :])
    p = ex / jnp.sum(ex, axis=0, keepdims=True)
    return p.T, ix.T.astype(jnp.int32)


def _router_body(xa_ref, xb_ref, w_ref, probs_ref, idx_ref):
    ha = xa_ref.shape[0]
    e = w_ref.shape[0]
    w = w_ref[...]
    iota = jax.lax.broadcasted_iota(jnp.int32, (e, _RC), 0).astype(jnp.float32)
    ef = float(e)
    n = 2 * ha // _RC
    npa = ha // _RC

    def dot_t(c):
        lo = (c - npa) * _RC if c >= npa else c * _RC
        ref = xb_ref if c >= npa else xa_ref
        return jax.lax.dot_general(
            w, ref[lo:lo + _RC, :], (((1,), (1,)), ((), ())),
            preferred_element_type=jnp.float32)

    st_next = dot_t(0)
    for c in range(n):
        st = st_next
        if c + 1 < n:
            st_next = dot_t(c + 1)
        p, ix = _topk_softmax_chunk_t(st, iota, ef)
        probs_ref[0, c * _RC:(c + 1) * _RC, :] = p
        idx_ref[0, c * _RC:(c + 1) * _RC, :] = ix


def kernel(x, W_gate):
    b, s, d = x.shape
    e = W_gate.shape[0]
    t = b * s
    bt = min(1024, s)
    grid = (t // bt,)
    spb = s // bt
    hw = bt // 2
    xf = x.reshape(t, d)
    probs, idx = pl.pallas_call(
        _router_body,
        grid=grid,
        in_specs=[
            pl.BlockSpec((hw, d), lambda i: (2 * i, 0)),
            pl.BlockSpec((hw, d), lambda i: (2 * i + 1, 0)),
            pl.BlockSpec((e, d), lambda i: (0, 0)),
        ],
        out_specs=[
            pl.BlockSpec((1, bt, _TOP_K), lambda i: (i // spb, i % spb, 0)),
            pl.BlockSpec((1, bt, _TOP_K), lambda i: (i // spb, i % spb, 0)),
        ],
        out_shape=[
            jax.ShapeDtypeStruct((b, s, _TOP_K), jnp.float32),
            jax.ShapeDtypeStruct((b, s, _TOP_K), jnp.int32),
        ],
        compiler_params=pltpu.CompilerParams(
            dimension_semantics=("parallel",)),
    )(xf, xf, W_gate)
    return probs, idx


# speedup vs baseline: 1.0726x; 1.0017x over previous
import jax
import jax.numpy as jnp
from jax.experimental import pallas as pl
from jax.experimental.pallas import tpu as pltpu

_TOP_K = 8
_RC = 128


def _topk_softmax_chunk_t(st, iota, ef):
    vals = []
    idxs = []
    for k in range(_TOP_K):
        m = jnp.max(st, axis=0, keepdims=True)
        eq = st == m
        hit = jnp.where(eq, iota, ef)
        idx = jnp.min(hit, axis=0, keepdims=True)
        vals.append(m)
        idxs.append(idx)
        if k + 1 < _TOP_K:
            st = jnp.where(eq, -jnp.inf, st)
    v = jnp.concatenate(vals, axis=0)
    ix = jnp.concatenate(idxs, axis=0)
    ex = jnp.exp(v - v[0:1, :])
    p = ex / jnp.sum(ex, axis=0, keepdims=True)
    return p.T, ix.T.astype(jnp.int32)


def _router_body(xa_ref, xb_ref, w_ref, probs_ref, idx_ref):
    ha = xa_ref.shape[0]
    e = w_ref.shape[0]
    w = w_ref[...]
    iota = jax.lax.broadcasted_iota(jnp.int32, (e, _RC), 0).astype(jnp.float32)
    ef = float(e)
    n = 2 * ha // _RC
    npa = ha // _RC

    def dot_t(c):
        lo = (c - npa) * _RC if c >= npa else c * _RC
        ref = xb_ref if c >= npa else xa_ref
        return jax.lax.dot_general(
            w, ref[lo:lo + _RC, :], (((1,), (1,)), ((), ())),
            preferred_element_type=jnp.float32)

    st_next = dot_t(0)
    for c in range(n):
        st = st_next
        if c + 1 < n:
            st_next = dot_t(c + 1)
        p, ix = _topk_softmax_chunk_t(st, iota, ef)
        probs_ref[0, c * _RC:(c + 1) * _RC, :] = p
        idx_ref[0, c * _RC:(c + 1) * _RC, :] = ix


def kernel(x, W_gate):
    b, s, d = x.shape
    e = W_gate.shape[0]
    t = b * s
    bt = min(1024, s)
    grid = (t // bt,)
    spb = s // bt
    hw = bt // 2
    xf = x.reshape(t, d)
    probs, idx = pl.pallas_call(
        _router_body,
        grid=grid,
        in_specs=[
            pl.BlockSpec((hw, d), lambda i: (2 * i, 0)),
            pl.BlockSpec((hw, d), lambda i: (2 * i + 1, 0)),
            pl.BlockSpec((e, d), lambda i: (0, 0)),
        ],
        out_specs=[
            pl.BlockSpec((1, bt, _TOP_K), lambda i: (i // spb, i % spb, 0)),
            pl.BlockSpec((1, bt, _TOP_K), lambda i: (i // spb, i % spb, 0)),
        ],
        out_shape=[
            jax.ShapeDtypeStruct((b, s, _TOP_K), jnp.float32),
            jax.ShapeDtypeStruct((b, s, _TOP_K), jnp.int32),
        ],
        compiler_params=pltpu.CompilerParams(
            dimension_semantics=("arbitrary",)),
    )(xf, xf, W_gate)
    return probs, idx
